# baseline (speedup 1.0000x reference)
import functools

import jax
import jax.numpy as jnp
from jax import lax
from jax.experimental import pallas as pl
from jax.experimental.pallas import tpu as pltpu
from jax.experimental.pallas import tpu_sc as plsc

_CH = 128
_SC = 256
_NBUF = 3


@functools.lru_cache(maxsize=None)
def _make_lookup(B, V, D, nc, ns):
    NW = nc * ns
    b_per_w = B // NW
    n_sc = b_per_w // _SC
    g_per_sc = _SC // _CH
    n_main = (n_sc // _NBUF) * _NBUF
    assert n_sc >= _NBUF
    mesh = plsc.VectorSubcoreMesh(core_axis_name="c", subcore_axis_name="s")

    @functools.partial(
        pl.kernel,
        mesh=mesh,
        out_type=jax.ShapeDtypeStruct((B, D), jnp.float32),
        scratch_types=[
            pltpu.VMEM_SHARED((V, D), jnp.float32),
            pltpu.VMEM((b_per_w,), jnp.int32),
            pltpu.VMEM((_NBUF, _SC, D), jnp.float32),
        ]
        + [pltpu.SemaphoreType.DMA] * (2 * _NBUF),
    )
    def lookup_kernel(idx_hbm, table_hbm, out_hbm, table_sh, idx_v, rows_v,
                      *sems):
        sem_g = sems[:_NBUF]
        sem_s = sems[_NBUF:]
        wid = lax.axis_index("s") * nc + lax.axis_index("c")
        base = wid * b_per_w

        @pl.when(lax.axis_index("s") == 0)
        def _():
            pltpu.sync_copy(table_hbm, table_sh)

        pltpu.sync_copy(idx_hbm.at[pl.ds(base, b_per_w)], idx_v)
        plsc.subcore_barrier()

        def gather_desc(s, b, h):
            idx_sl = idx_v.at[pl.ds(s * _SC + h * _CH, _CH)]
            return pltpu.make_async_copy(
                table_sh.at[idx_sl],
                rows_v.at[b].at[pl.ds(h * _CH, _CH)],
                sem_g[b])

        def scatter_desc(s, b):
            return pltpu.make_async_copy(
                rows_v.at[b],
                out_hbm.at[pl.ds(base + s * _SC, _SC)],
                sem_s[b])

        def _maybe_when(cond, fn):
            if isinstance(cond, bool):
                if cond:
                    fn()
            else:
                pl.when(cond)(fn)

        def step(s, b, nb, wait_cond, prefetch_cond):
            _maybe_when(wait_cond, lambda: scatter_desc(s - 2, nb).wait())

            def _prefetch():
                for h in range(g_per_sc):
                    gather_desc(s + 1, nb, h).start()

            _maybe_when(prefetch_cond, _prefetch)

            for h in range(g_per_sc):
                gather_desc(s, b, h).wait()
            scatter_desc(s, b).start()

        for h in range(g_per_sc):
            gather_desc(0, 0, h).start()

        def body(ss, carry):
            for u in range(_NBUF):
                s = ss * _NBUF + u
                step(s, u, (u + 1) % _NBUF, s >= 2, s + 1 < n_sc)
            return carry

        lax.fori_loop(0, n_main // _NBUF, body, 0)
        for s in range(n_main, n_sc):
            step(s, s % _NBUF, (s + 1) % _NBUF, s >= 2, s + 1 < n_sc)
        scatter_desc(n_sc - 2, (n_sc - 2) % _NBUF).wait()
        scatter_desc(n_sc - 1, (n_sc - 1) % _NBUF).wait()

    return lookup_kernel


def kernel(elems, table):
    shape = elems.shape
    V, D = table.shape
    idx = elems.reshape(-1).astype(jnp.int32)
    B = idx.shape[0]
    info = plsc.get_sparse_core_info()
    nc, ns = info.num_cores, info.num_subcores
    group = nc * ns * _SC
    Bp = ((B + group - 1) // group) * group
    if Bp != B:
        idx = jnp.pad(idx, (0, Bp - B))
    out = _make_lookup(Bp, V, D, nc, ns)(idx, table)
    if Bp != B:
        out = out[:B]
    return out.reshape(*shape, D)


# speedup vs baseline: 1.0659x; 1.0659x over previous
import functools

import jax
import jax.numpy as jnp
from jax import lax
from jax.experimental import pallas as pl
from jax.experimental.pallas import tpu as pltpu
from jax.experimental.pallas import tpu_sc as plsc

_CH = 128
_NBUF = 5
_LA = 3


@functools.lru_cache(maxsize=None)
def _make_lookup(B, V, D, nc, ns):
    NW = nc * ns
    b_per_w = B // NW
    n_chunks = b_per_w // _CH
    assert n_chunks % _NBUF == 0 and n_chunks >= _NBUF
    mesh = plsc.VectorSubcoreMesh(core_axis_name="c", subcore_axis_name="s")

    @functools.partial(
        pl.kernel,
        mesh=mesh,
        out_type=jax.ShapeDtypeStruct((B, D), jnp.float32),
        scratch_types=[
            pltpu.VMEM_SHARED((V, D), jnp.float32),
            pltpu.VMEM((b_per_w,), jnp.int32),
            pltpu.VMEM((_NBUF, _CH, D), jnp.float32),
        ]
        + [pltpu.SemaphoreType.DMA] * (2 * _NBUF),
    )
    def lookup_kernel(idx_hbm, table_hbm, out_hbm, table_sh, idx_v, rows_v,
                      *sems):
        sem_g = sems[:_NBUF]
        sem_s = sems[_NBUF:]
        wid = lax.axis_index("s") * nc + lax.axis_index("c")
        base = wid * b_per_w

        @pl.when(lax.axis_index("s") == 0)
        def _():
            pltpu.sync_copy(table_hbm, table_sh)

        pltpu.sync_copy(idx_hbm.at[pl.ds(base, b_per_w)], idx_v)
        plsc.subcore_barrier()

        def gather_desc(g, b):
            idx_sl = idx_v.at[pl.ds(g * _CH, _CH)]
            return pltpu.make_async_copy(
                table_sh.at[idx_sl], rows_v.at[b], sem_g[b])

        def scatter_desc(g, b):
            return pltpu.make_async_copy(
                rows_v.at[b],
                out_hbm.at[pl.ds(base + g * _CH, _CH)],
                sem_s[b])

        for c in range(_LA):
            gather_desc(c, c).start()

        def body(gg, carry):
            for b in range(_NBUF):
                g = gg * _NBUF + b
                bg = (b + _LA) % _NBUF

                @pl.when(g >= _NBUF - _LA)
                def _():
                    scatter_desc(g + _LA - _NBUF, bg).wait()

                @pl.when(g + _LA < n_chunks)
                def _():
                    gather_desc(g + _LA, bg).start()

                gather_desc(g, b).wait()
                scatter_desc(g, b).start()
            return carry

        lax.fori_loop(0, n_chunks // _NBUF, body, 0)
        for c in range(n_chunks - _NBUF + _LA, n_chunks):
            scatter_desc(c, c % _NBUF).wait()

    return lookup_kernel


def kernel(elems, table):
    shape = elems.shape
    V, D = table.shape
    idx = elems.reshape(-1).astype(jnp.int32)
    B = idx.shape[0]
    info = plsc.get_sparse_core_info()
    nc, ns = info.num_cores, info.num_subcores
    group = nc * ns * _CH * _NBUF
    Bp = ((B + group - 1) // group) * group
    if Bp != B:
        idx = jnp.pad(idx, (0, Bp - B))
    out = _make_lookup(Bp, V, D, nc, ns)(idx, table)
    if Bp != B:
        out = out[:B]
    return out.reshape(*shape, D)


# speedup vs baseline: 1.0717x; 1.0055x over previous
import functools

import jax
import jax.numpy as jnp
from jax import lax
from jax.experimental import pallas as pl
from jax.experimental.pallas import tpu as pltpu
from jax.experimental.pallas import tpu_sc as plsc

_CH = 64
_NBUF = 10
_LA = 6


@functools.lru_cache(maxsize=None)
def _make_lookup(B, V, D, nc, ns):
    NW = nc * ns
    b_per_w = B // NW
    n_chunks = b_per_w // _CH
    assert n_chunks % _NBUF == 0 and n_chunks >= _NBUF
    mesh = plsc.VectorSubcoreMesh(core_axis_name="c", subcore_axis_name="s")

    @functools.partial(
        pl.kernel,
        mesh=mesh,
        out_type=jax.ShapeDtypeStruct((B, D), jnp.float32),
        scratch_types=[
            pltpu.VMEM_SHARED((V, D), jnp.float32),
            pltpu.VMEM((b_per_w,), jnp.int32),
            pltpu.VMEM((_NBUF, _CH, D), jnp.float32),
        ]
        + [pltpu.SemaphoreType.DMA] * (2 * _NBUF),
    )
    def lookup_kernel(idx_hbm, table_hbm, out_hbm, table_sh, idx_v, rows_v,
                      *sems):
        sem_g = sems[:_NBUF]
        sem_s = sems[_NBUF:]
        wid = lax.axis_index("s") * nc + lax.axis_index("c")
        base = wid * b_per_w

        @pl.when(lax.axis_index("s") == 0)
        def _():
            pltpu.sync_copy(table_hbm, table_sh)

        pltpu.sync_copy(idx_hbm.at[pl.ds(base, b_per_w)], idx_v)
        plsc.subcore_barrier()

        def gather_desc(g, b):
            idx_sl = idx_v.at[pl.ds(g * _CH, _CH)]
            return pltpu.make_async_copy(
                table_sh.at[idx_sl], rows_v.at[b], sem_g[b])

        def scatter_desc(g, b):
            return pltpu.make_async_copy(
                rows_v.at[b],
                out_hbm.at[pl.ds(base + g * _CH, _CH)],
                sem_s[b])

        for c in range(_LA):
            gather_desc(c, c).start()

        def body(gg, carry):
            for b in range(_NBUF):
                g = gg * _NBUF + b
                bg = (b + _LA) % _NBUF

                @pl.when(g >= _NBUF - _LA)
                def _():
                    scatter_desc(g + _LA - _NBUF, bg).wait()

                @pl.when(g + _LA < n_chunks)
                def _():
                    gather_desc(g + _LA, bg).start()

                gather_desc(g, b).wait()
                scatter_desc(g, b).start()
            return carry

        lax.fori_loop(0, n_chunks // _NBUF, body, 0)
        for c in range(n_chunks - _NBUF + _LA, n_chunks):
            scatter_desc(c, c % _NBUF).wait()

    return lookup_kernel


def kernel(elems, table):
    shape = elems.shape
    V, D = table.shape
    idx = elems.reshape(-1).astype(jnp.int32)
    B = idx.shape[0]
    info = plsc.get_sparse_core_info()
    nc, ns = info.num_cores, info.num_subcores
    group = nc * ns * _CH * _NBUF
    Bp = ((B + group - 1) // group) * group
    if Bp != B:
        idx = jnp.pad(idx, (0, Bp - B))
    out = _make_lookup(Bp, V, D, nc, ns)(idx, table)
    if Bp != B:
        out = out[:B]
    return out.reshape(*shape, D)
